# initial kernel scaffold (unmeasured)
import jax
import jax.numpy as jnp
from jax import lax
from jax.experimental import pallas as pl
from jax.experimental.pallas import tpu as pltpu

N_DEV = 4
N_LAYERS = 3


def kernel(x, Win0, Wout0, Win1, Wout1, Win2, Wout2):
    B, D = x.shape
    rows_out = B // N_DEV

    def body(x_ref, win0_ref, wout0_ref, win1_ref, wout1_ref, win2_ref,
             wout2_ref, out_ref, comm_ref, send_sems, recv_sems, bar_sems):
        my = lax.axis_index("i")
        left = (my + N_DEV - 1) % N_DEV
        right = (my + 1) % N_DEV

        wins = [win0_ref, win1_ref, win2_ref]
        wouts = [wout0_ref, wout1_ref, wout2_ref]

        xv = x_ref[:, :]
        for layer in range(N_LAYERS):
            h = jnp.maximum(
                jnp.dot(xv, wins[layer][:, :],
                        preferred_element_type=jnp.float32),
                0.0,
            )
            partial = jnp.dot(h, wouts[layer][:, :],
                              preferred_element_type=jnp.float32)

            comm_ref[0, :, :] = partial
            for hop in range(N_DEV - 1):
                rdma = pltpu.make_async_remote_copy(
                    src_ref=comm_ref.at[hop],
                    dst_ref=comm_ref.at[hop + 1],
                    send_sem=send_sems.at[hop],
                    recv_sem=recv_sems.at[hop + 1],
                    device_id=(right,),
                    device_id_type=pl.DeviceIdType.MESH,
                )
                rdma.start()
                rdma.wait()

            total = (comm_ref[0, :, :] + comm_ref[1, :, :]) + \
                    (comm_ref[2, :, :] + comm_ref[3, :, :])

            pl.semaphore_signal(bar_sems.at[layer], inc=1, device_id=(left,),
                                device_id_type=pl.DeviceIdType.MESH)
            pl.semaphore_signal(bar_sems.at[layer], inc=1, device_id=(right,),
                                device_id_type=pl.DeviceIdType.MESH)
            pl.semaphore_wait(bar_sems.at[layer], 2)

            if layer < N_LAYERS - 1:
                xv = total
            else:
                out_ref[:, :] = lax.dynamic_slice_in_dim(
                    total, my * rows_out, rows_out, 0)

    return pl.pallas_call(
        body,
        out_shape=jax.ShapeDtypeStruct((rows_out, D), jnp.float32),
        in_specs=[pl.BlockSpec(memory_space=pltpu.VMEM)] * 7,
        out_specs=pl.BlockSpec(memory_space=pltpu.VMEM),
        scratch_shapes=[
            pltpu.VMEM((N_DEV, B, D), jnp.float32),
            pltpu.SemaphoreType.DMA((N_DEV - 1,)),
            pltpu.SemaphoreType.DMA((N_DEV,)),
            pltpu.SemaphoreType.REGULAR((N_LAYERS,)),
        ],
    )(x, Win0, Wout0, Win1, Wout1, Win2, Wout2)


# baseline (device time: 84270 ns/iter reference)
import jax
import jax.numpy as jnp
from jax import lax
from jax.experimental import pallas as pl
from jax.experimental.pallas import tpu as pltpu

N_DEV = 4
N_LAYERS = 3


def kernel(x, Win0, Wout0, Win1, Wout1, Win2, Wout2):
    B, D = x.shape
    rows_out = B // N_DEV

    def body(x_ref, win0_ref, wout0_ref, win1_ref, wout1_ref, win2_ref,
             wout2_ref, out_ref, comm_ref, send_sems, recv_sems, bar_sems):
        my = lax.axis_index("i")
        left = (my + N_DEV - 1) % N_DEV
        right = (my + 1) % N_DEV

        wins = [win0_ref, win1_ref, win2_ref]
        wouts = [wout0_ref, wout1_ref, wout2_ref]

        xv = x_ref[:, :]
        for layer in range(N_LAYERS):
            h = jnp.maximum(
                jnp.dot(xv, wins[layer][:, :],
                        preferred_element_type=jnp.float32),
                0.0,
            )
            partial = jnp.dot(h, wouts[layer][:, :],
                              preferred_element_type=jnp.float32)

            comm_ref[0, :, :] = partial
            for hop in range(N_DEV - 1):
                rdma = pltpu.make_async_remote_copy(
                    src_ref=comm_ref.at[hop],
                    dst_ref=comm_ref.at[hop + 1],
                    send_sem=send_sems.at[hop],
                    recv_sem=recv_sems.at[hop + 1],
                    device_id=(right,),
                    device_id_type=pl.DeviceIdType.MESH,
                )
                rdma.start()
                rdma.wait()

            if layer < N_LAYERS - 1:
                xv = (comm_ref[0, :, :] + comm_ref[1, :, :]) + \
                     (comm_ref[2, :, :] + comm_ref[3, :, :])
            else:
                r0 = pl.ds(my * rows_out, rows_out)
                out_ref[:, :] = (comm_ref[0, r0, :] + comm_ref[1, r0, :]) + \
                                (comm_ref[2, r0, :] + comm_ref[3, r0, :])

            pl.semaphore_signal(bar_sems.at[layer], inc=1, device_id=(left,),
                                device_id_type=pl.DeviceIdType.MESH)
            pl.semaphore_signal(bar_sems.at[layer], inc=1, device_id=(right,),
                                device_id_type=pl.DeviceIdType.MESH)
            pl.semaphore_wait(bar_sems.at[layer], 2)

    return pl.pallas_call(
        body,
        out_shape=jax.ShapeDtypeStruct((rows_out, D), jnp.float32),
        in_specs=[pl.BlockSpec(memory_space=pltpu.VMEM)] * 7,
        out_specs=pl.BlockSpec(memory_space=pltpu.VMEM),
        scratch_shapes=[
            pltpu.VMEM((N_DEV, B, D), jnp.float32),
            pltpu.SemaphoreType.DMA((N_DEV - 1,)),
            pltpu.SemaphoreType.DMA((N_DEV,)),
            pltpu.SemaphoreType.REGULAR((N_LAYERS,)),
        ],
    )(x, Win0, Wout0, Win1, Wout1, Win2, Wout2)


# device time: 38257 ns/iter; 2.2027x vs baseline; 2.2027x over previous
import jax
import jax.numpy as jnp
from jax import lax
from jax.experimental import pallas as pl
from jax.experimental.pallas import tpu as pltpu

N_DEV = 4
N_LAYERS = 3
RS, AG = 0, 1


def kernel(x, Win0, Wout0, Win1, Wout1, Win2, Wout2):
    B, D = x.shape
    R = B // N_DEV

    def body(x_ref, win0_ref, wout0_ref, win1_ref, wout1_ref, win2_ref,
             wout2_ref, out_ref, part_ref, rs_ref, ag_ref,
             send_sems, recv_sems):
        my = lax.axis_index("i")
        wins = [win0_ref, win1_ref, win2_ref]
        wouts = [wout0_ref, wout1_ref, wout2_ref]

        pending = []

        def remote(src, dst, l, phase, send_slot, recv_slot, target):
            return pltpu.make_async_remote_copy(
                src_ref=src, dst_ref=dst,
                send_sem=send_sems.at[l, phase, send_slot],
                recv_sem=recv_sems.at[l, phase, recv_slot],
                device_id=(target,),
                device_id_type=pl.DeviceIdType.MESH,
            )

        xv = x_ref[:, :]
        for l in range(N_LAYERS):
            h = jnp.maximum(
                jnp.dot(xv, wins[l][:, :], preferred_element_type=jnp.float32),
                0.0)
            partial = jnp.dot(h, wouts[l][:, :],
                              preferred_element_type=jnp.float32)
            part_ref[:, :] = partial

            for o in (2, 1, 3):
                e = (my + o) % N_DEV
                rdma = remote(part_ref.at[pl.ds(e * R, R), :], rs_ref.at[my],
                              l, RS, e, my, e)
                rdma.start()
                pending.append(rdma)
            rs_ref[my, :, :] = part_ref[pl.ds(my * R, R), :]
            for o in (1, 2, 3):
                s = (my + o) % N_DEV
                remote(part_ref.at[pl.ds(0, R), :], rs_ref.at[s],
                       l, RS, s, s, s).wait_recv()
            total = (rs_ref[0, :, :] + rs_ref[1, :, :]) + \
                    (rs_ref[2, :, :] + rs_ref[3, :, :])

            if l < N_LAYERS - 1:
                ag_ref[my, :, :] = total
                for o in (2, 1, 3):
                    e = (my + o) % N_DEV
                    rdma = remote(ag_ref.at[my], ag_ref.at[my], l, AG, e, my, e)
                    rdma.start()
                    pending.append(rdma)
                for o in (1, 2, 3):
                    c = (my + o) % N_DEV
                    remote(ag_ref.at[my], ag_ref.at[c],
                           l, AG, c, c, c).wait_recv()
                xv = jnp.concatenate(
                    [ag_ref[0, :, :], ag_ref[1, :, :],
                     ag_ref[2, :, :], ag_ref[3, :, :]], axis=0)
            else:
                out_ref[:, :] = total

        for rdma in pending:
            rdma.wait_send()

    return pl.pallas_call(
        body,
        out_shape=jax.ShapeDtypeStruct((R, D), jnp.float32),
        in_specs=[pl.BlockSpec(memory_space=pltpu.VMEM)] * 7,
        out_specs=pl.BlockSpec(memory_space=pltpu.VMEM),
        scratch_shapes=[
            pltpu.VMEM((B, D), jnp.float32),
            pltpu.VMEM((N_DEV, R, D), jnp.float32),
            pltpu.VMEM((N_DEV, R, D), jnp.float32),
            pltpu.SemaphoreType.DMA((N_LAYERS, 2, N_DEV)),
            pltpu.SemaphoreType.DMA((N_LAYERS, 2, N_DEV)),
        ],
    )(x, Win0, Wout0, Win1, Wout1, Win2, Wout2)
